# baseline (speedup 1.0000x reference)
import functools

import jax
import jax.numpy as jnp
import numpy as np
from jax import lax
from jax.experimental import pallas as pl
from jax.experimental.pallas import tpu as pltpu
from jax.experimental.pallas import tpu_sc as plsc

D = 64
L = 16
NC, NS = 2, 16
NW = NC * NS
BLK = 128
EPS = 1e-5
UNROLL = 4
NEWTON = 1


def _ln_impl(idxt, table, gamma, beta):
  nh, nb = idxt.shape
  ntb = nb // BLK
  assert ntb == NW

  mesh = plsc.VectorSubcoreMesh(
      core_axis_name="c", subcore_axis_name="s", num_cores=NC, num_subcores=NS
  )

  @functools.partial(
      pl.kernel,
      out_type=jax.ShapeDtypeStruct((nh, D // 8, ntb, 8 * BLK), jnp.float32),
      mesh=mesh,
      compiler_params=pltpu.CompilerParams(
          use_tc_tiling_on_sc=False, needs_layout_passes=False),
      scratch_types=[
          pltpu.VMEM((2, BLK), jnp.int32),
          pltpu.VMEM((2, BLK, D), jnp.float32),
          pltpu.VMEM((D * BLK,), jnp.float32),
          pltpu.VMEM((D * BLK,), jnp.float32),
          pltpu.VMEM((D,), jnp.float32),
          pltpu.VMEM((D,), jnp.float32),
          pltpu.SemaphoreType.DMA,
          pltpu.SemaphoreType.DMA,
          pltpu.SemaphoreType.DMA,
          pltpu.SemaphoreType.DMA,
      ],
  )
  def k(idx_hbm, table_hbm, gamma_hbm, beta_hbm, out_hbm, idxv, ibuf, obuf0,
        obuf1, gamma_v, beta_v, gsem0, gsem1, osem0, osem1):
    obufs = [obuf0, obuf1]
    wid = lax.axis_index("s") * NC + lax.axis_index("c")

    pltpu.sync_copy(gamma_hbm, gamma_v)
    pltpu.sync_copy(beta_hbm, beta_v)
    gvec = [gamma_v[pl.ds(L * t, L)] for t in range(D // L)]
    bvec = [beta_v[pl.ds(L * t, L)] for t in range(D // L)]
    lane = lax.iota(jnp.int32, L)
    perms = [(lane + sh) & (L - 1) for sh in (8, 4, 2, 1)]
    posvec = [lax.shift_left(lane + L * t, 7) for t in range(D // L)]

    def fire_task(h, p, gsem):
      pltpu.sync_copy(idx_hbm.at[h, pl.ds(wid * BLK, BLK)], idxv.at[p])
      pltpu.async_copy(table_hbm.at[idxv.at[p]], ibuf.at[p], gsem)

    def wait_gather(p, gsem):
      pltpu.make_async_copy(table_hbm.at[idxv.at[p]], ibuf.at[p], gsem).wait()

    def compute(p):
      def row4(rr, carry):
        for u in range(UNROLL):
          r = rr * UNROLL + u
          x = [ibuf[p, r, pl.ds(L * t, L)] for t in range(D // L)]
          s = (x[0] + x[1]) + (x[2] + x[3])
          sq = (x[0] * x[0] + x[1] * x[1]) + (x[2] * x[2] + x[3] * x[3])
          for pm in perms:
            s = s + s.at[pm].get(mode="promise_in_bounds")
            sq = sq + sq.at[pm].get(mode="promise_in_bounds")
          mean_v = s * (1.0 / D)
          ex2 = sq * (1.0 / D)
          tv = ex2 - mean_v * mean_v + EPS
          seed = lax.bitcast_convert_type(tv, jnp.int32)
          seed = 0x5F3759DF - lax.shift_right_logical(seed, 1)
          g = lax.bitcast_convert_type(seed, jnp.float32)
          htv = 0.5 * tv
          for _ in range(NEWTON):
            g = g * (1.5 - htv * g * g)
          rfull = jnp.full((L,), r, jnp.int32)
          for t in range(D // L):
            y = (x[t] - mean_v) * g * gvec[t] + bvec[t]
            plsc.store_scatter(obufs[p], [posvec[t] + rfull], y)
        return carry

      lax.fori_loop(0, BLK // UNROLL, row4, 0)

    def fire_out(h, p, osem):
      for td in range(D // 8):
        pltpu.async_copy(obufs[p].at[pl.ds(td * 8 * BLK, 8 * BLK)],
                         out_hbm.at[h, td, wid], osem)

    def wait_out(p, osem):
      for td in range(D // 8):
        pltpu.make_async_copy(obufs[p].at[pl.ds(td * 8 * BLK, 8 * BLK)],
                              out_hbm.at[0, td, wid], osem).wait()

    fire_task(0, 0, gsem0)

    def pair_body(i, carry):
      ha = 2 * i
      fire_task(ha + 1, 1, gsem1)
      wait_gather(0, gsem0)

      @pl.when(i > 0)
      def _():
        wait_out(0, osem0)

      compute(0)
      fire_out(ha, 0, osem0)

      @pl.when(i < nh // 2 - 1)
      def _():
        fire_task(ha + 2, 0, gsem0)

      wait_gather(1, gsem1)

      @pl.when(i > 0)
      def _():
        wait_out(1, osem1)

      compute(1)
      fire_out(ha + 1, 1, osem1)
      return carry

    lax.fori_loop(0, nh // 2, pair_body, 0)
    wait_out(0, osem0)
    wait_out(1, osem1)

  return k(idxt, table, gamma, beta)


def kernel(job_id, table, gamma, beta):
  b, h = job_id.shape
  idxt = job_id.T.astype(jnp.int32)
  out5 = _ln_impl(idxt, table, gamma, beta)
  out = (out5.reshape(h, D // 8, b // BLK, 8, BLK)
         .transpose(2, 4, 0, 1, 3).reshape(b, h, D))
  return out


# speedup vs baseline: 1.0472x; 1.0472x over previous
import functools

import jax
import jax.numpy as jnp
from jax import lax
from jax.experimental import pallas as pl
from jax.experimental.pallas import tpu as pltpu
from jax.experimental.pallas import tpu_sc as plsc

D = 64
L = 16
NC, NS = 2, 16
NW = NC * NS
BLK = 128
EPS = 1e-5
UNROLL = 4
NEWTON = 1


def _ln_impl(idxt, table, gamma, beta):
  nh, nb = idxt.shape
  ntb = nb // BLK
  assert ntb == NW

  mesh = plsc.VectorSubcoreMesh(
      core_axis_name="c", subcore_axis_name="s", num_cores=NC, num_subcores=NS
  )

  @functools.partial(
      pl.kernel,
      out_type=jax.ShapeDtypeStruct((nh, D // 8, ntb, 8 * BLK), jnp.float32),
      mesh=mesh,
      compiler_params=pltpu.CompilerParams(
          use_tc_tiling_on_sc=False, needs_layout_passes=False),
      scratch_types=[
          pltpu.VMEM((nh, BLK), jnp.int32),
          pltpu.VMEM((2, BLK, D), jnp.float32),
          pltpu.VMEM((D // 8, 8 * BLK), jnp.float32),
          pltpu.VMEM((D // 8, 8 * BLK), jnp.float32),
          pltpu.VMEM((D,), jnp.float32),
          pltpu.VMEM((D,), jnp.float32),
          pltpu.SemaphoreType.DMA,
          pltpu.SemaphoreType.DMA,
          pltpu.SemaphoreType.DMA,
          pltpu.SemaphoreType.DMA,
      ],
  )
  def k(idx_hbm, table_hbm, gamma_hbm, beta_hbm, out_hbm, idxall, ibuf,
        obuf0, obuf1, gamma_v, beta_v, gsem0, gsem1, osem0, osem1):
    obufs = [obuf0, obuf1]
    wid = lax.axis_index("s") * NC + lax.axis_index("c")

    pltpu.sync_copy(gamma_hbm, gamma_v)
    pltpu.sync_copy(beta_hbm, beta_v)
    pltpu.sync_copy(idx_hbm.at[:, pl.ds(wid * BLK, BLK)], idxall)

    gvec = [gamma_v[pl.ds(L * t, L)] for t in range(D // L)]
    bvec = [beta_v[pl.ds(L * t, L)] for t in range(D // L)]
    lane = lax.iota(jnp.int32, L)
    perms = [(lane + sh) & (L - 1) for sh in (8, 4, 2, 1)]
    dvals = [lane + L * t for t in range(D // L)]
    tdvec = [lax.shift_right_logical(dv, 3) for dv in dvals]
    posvec = [lax.shift_left(dv & 7, 7) for dv in dvals]

    def fire_gather(h, p, gsem):
      pltpu.async_copy(table_hbm.at[idxall.at[h]], ibuf.at[p], gsem)

    def wait_gather(p, gsem):
      pltpu.make_async_copy(table_hbm.at[idxall.at[0]], ibuf.at[p],
                            gsem).wait()

    def compute(p):
      obuf = obufs[p]

      def row4(rr, carry):
        for u in range(UNROLL):
          r = rr * UNROLL + u
          x = [ibuf[p, r, pl.ds(L * t, L)] for t in range(D // L)]
          s = (x[0] + x[1]) + (x[2] + x[3])
          sq = (x[0] * x[0] + x[1] * x[1]) + (x[2] * x[2] + x[3] * x[3])
          for pm in perms:
            s = s + s.at[pm].get(mode="promise_in_bounds")
            sq = sq + sq.at[pm].get(mode="promise_in_bounds")
          mean_v = s * (1.0 / D)
          ex2 = sq * (1.0 / D)
          tv = ex2 - mean_v * mean_v + EPS
          seed = lax.bitcast_convert_type(tv, jnp.int32)
          seed = 0x5F3759DF - lax.shift_right_logical(seed, 1)
          g = lax.bitcast_convert_type(seed, jnp.float32)
          htv = 0.5 * tv
          for _ in range(NEWTON):
            g = g * (1.5 - htv * g * g)
          rfull = jnp.full((L,), r, jnp.int32)
          for t in range(D // L):
            y = (x[t] - mean_v) * g * gvec[t] + bvec[t]
            plsc.store_scatter(obuf, [tdvec[t], posvec[t] + rfull], y)
        return carry

      lax.fori_loop(0, BLK // UNROLL, row4, 0)

    def fire_out(h, p, osem):
      pltpu.async_copy(obufs[p], out_hbm.at[h, :, wid], osem)

    def wait_out(p, osem):
      pltpu.make_async_copy(obufs[p], out_hbm.at[0, :, wid], osem).wait()

    fire_gather(0, 0, gsem0)

    def pair_body(i, carry):
      ha = 2 * i
      fire_gather(ha + 1, 1, gsem1)
      wait_gather(0, gsem0)

      @pl.when(i > 0)
      def _():
        wait_out(0, osem0)

      compute(0)
      fire_out(ha, 0, osem0)

      @pl.when(i < nh // 2 - 1)
      def _():
        fire_gather(ha + 2, 0, gsem0)

      wait_gather(1, gsem1)

      @pl.when(i > 0)
      def _():
        wait_out(1, osem1)

      compute(1)
      fire_out(ha + 1, 1, osem1)
      return carry

    lax.fori_loop(0, nh // 2, pair_body, 0)
    wait_out(0, osem0)
    wait_out(1, osem1)

  return k(idxt, table, gamma, beta)


def kernel(job_id, table, gamma, beta):
  b, h = job_id.shape
  idxt = job_id.T.astype(jnp.int32)
  out5 = _ln_impl(idxt, table, gamma, beta)
  out = (out5.reshape(h, D // 8, b // BLK, 8, BLK)
         .transpose(2, 4, 0, 1, 3).reshape(b, h, D))
  return out


# speedup vs baseline: 1.5613x; 1.4909x over previous
import functools

import jax
import jax.numpy as jnp
from jax import lax
from jax.experimental import pallas as pl
from jax.experimental.pallas import tpu as pltpu
from jax.experimental.pallas import tpu_sc as plsc

D = 64
L = 16
NC, NS = 2, 16
NW = NC * NS
BLK = 128
EPS = 1e-5
UNROLL = 4
NEWTON = 1


def _ln_impl(idxt, table, gamma, beta):
  nh, nb = idxt.shape
  ntb = nb // BLK
  assert ntb == NW

  mesh = plsc.VectorSubcoreMesh(
      core_axis_name="c", subcore_axis_name="s", num_cores=NC, num_subcores=NS
  )

  @functools.partial(
      pl.kernel,
      out_type=jax.ShapeDtypeStruct((nh, D // 8, ntb, 8, BLK), jnp.float32),
      mesh=mesh,
      compiler_params=pltpu.CompilerParams(
          use_tc_tiling_on_sc=False, needs_layout_passes=False),
      scratch_types=[
          pltpu.VMEM((nh, BLK), jnp.int32),
          pltpu.VMEM((2, BLK, 2 * D), jnp.float32),
          pltpu.VMEM((D // 8, 8, BLK + 1), jnp.float32),
          pltpu.VMEM((D // 8, 8, BLK + 1), jnp.float32),
          pltpu.VMEM((D,), jnp.float32),
          pltpu.VMEM((D,), jnp.float32),
          pltpu.SemaphoreType.DMA,
          pltpu.SemaphoreType.DMA,
          pltpu.SemaphoreType.DMA,
          pltpu.SemaphoreType.DMA,
      ],
  )
  def k(idx_hbm, table_hbm, gamma_hbm, beta_hbm, out_hbm, idxall, ibuf,
        obuf0, obuf1, gamma_v, beta_v, gsem0, gsem1, osem0, osem1):
    obufs = [obuf0, obuf1]
    wid = lax.axis_index("s") * NC + lax.axis_index("c")

    pltpu.sync_copy(gamma_hbm, gamma_v)
    pltpu.sync_copy(beta_hbm, beta_v)
    pltpu.sync_copy(idx_hbm.at[:, pl.ds(wid * BLK, BLK)], idxall)

    gvec = [gamma_v[pl.ds(L * t, L)] for t in range(D // L)]
    bvec = [beta_v[pl.ds(L * t, L)] for t in range(D // L)]
    lane = lax.iota(jnp.int32, L)
    perms = [(lane + sh) & (L - 1) for sh in (8, 4, 2, 1)]
    dvals = [lane + L * t for t in range(D // L)]
    tdvec = [lax.shift_right_logical(dv, 3) for dv in dvals]
    d8vec = [dv & 7 for dv in dvals]

    def fire_gather(h, p, gsem):
      pltpu.async_copy(table_hbm.at[idxall.at[h]], ibuf.at[p], gsem)

    def wait_gather(p, gsem):
      pltpu.make_async_copy(table_hbm.at[idxall.at[0]], ibuf.at[p],
                            gsem).wait()

    def compute(p):
      obuf = obufs[p]

      def row4(rr, carry):
        for u in range(UNROLL):
          r = rr * UNROLL + u
          x = [ibuf[p, r, pl.ds(L * t, L)] for t in range(D // L)]
          s = (x[0] + x[1]) + (x[2] + x[3])
          sq = (x[0] * x[0] + x[1] * x[1]) + (x[2] * x[2] + x[3] * x[3])
          for pm in perms:
            s = s + s.at[pm].get(mode="promise_in_bounds")
            sq = sq + sq.at[pm].get(mode="promise_in_bounds")
          mean_v = s * (1.0 / D)
          ex2 = sq * (1.0 / D)
          tv = ex2 - mean_v * mean_v + EPS
          seed = lax.bitcast_convert_type(tv, jnp.int32)
          seed = 0x5F3759DF - lax.shift_right_logical(seed, 1)
          g = lax.bitcast_convert_type(seed, jnp.float32)
          htv = 0.5 * tv
          for _ in range(NEWTON):
            g = g * (1.5 - htv * g * g)
          rfull = jnp.full((L,), r, jnp.int32)
          for t in range(D // L):
            y = (x[t] - mean_v) * g * gvec[t] + bvec[t]
            plsc.store_scatter(obuf, [tdvec[t], d8vec[t], rfull], y)
        return carry

      lax.fori_loop(0, BLK // UNROLL, row4, 0)

    def fire_out(h, p, osem):
      pltpu.async_copy(obufs[p].at[:, :, pl.ds(0, BLK)], out_hbm.at[h, :, wid],
                       osem)

    def wait_out(p, osem):
      pltpu.make_async_copy(obufs[p].at[:, :, pl.ds(0, BLK)],
                            out_hbm.at[0, :, wid], osem).wait()

    fire_gather(0, 0, gsem0)

    def pair_body(i, carry):
      ha = 2 * i
      fire_gather(ha + 1, 1, gsem1)
      wait_gather(0, gsem0)

      @pl.when(i > 0)
      def _():
        wait_out(0, osem0)

      compute(0)
      fire_out(ha, 0, osem0)

      @pl.when(i < nh // 2 - 1)
      def _():
        fire_gather(ha + 2, 0, gsem0)

      wait_gather(1, gsem1)

      @pl.when(i > 0)
      def _():
        wait_out(1, osem1)

      compute(1)
      fire_out(ha + 1, 1, osem1)
      return carry

    lax.fori_loop(0, nh // 2, pair_body, 0)
    wait_out(0, osem0)
    wait_out(1, osem1)

  return k(idxt, table, gamma, beta)


def kernel(job_id, table, gamma, beta):
  b, h = job_id.shape
  idxt = job_id.T.astype(jnp.int32)
  table_p = jnp.pad(table, ((0, 0), (0, D)))
  out5 = _ln_impl(idxt, table_p, gamma, beta)
  out = out5.transpose(2, 4, 0, 1, 3).reshape(b, h, D)
  return out
